# baseline (speedup 1.0000x reference)
import functools

import jax
import jax.numpy as jnp
from jax import lax
from jax.experimental import pallas as pl
from jax.experimental.pallas import tpu as pltpu
from jax.experimental.pallas import tpu_sc as plsc

N_NODES = 10000
N_EDGES = 160000
D_FEAT = 128
D_EDGE = 4
R_DIM = 8
HID = 64
N_IRR = D_FEAT * (1 + D_EDGE)

NC = 2
NS = 16
NW = NC * NS

EBLK = 128
NBLK = N_EDGES // EBLK
CCH = 128
DRAIN_ROWS = 632

GBLK_PER_TILE = 40
E_PAD = 32 * GBLK_PER_TILE * EBLK

SBLK_PER_TILE = 78
RCV_PAD_BLKS = 1344

EB_TC = 1280


def _sc_gather(node_feats, senders2d):
  mesh = plsc.VectorSubcoreMesh(core_axis_name="c", subcore_axis_name="s")

  @functools.partial(
      pl.kernel,
      out_type=jax.ShapeDtypeStruct((E_PAD, D_FEAT), jnp.float32),
      mesh=mesh,
      scratch_types=[
          pltpu.VMEM((GBLK_PER_TILE, EBLK), jnp.int32),
          pltpu.VMEM((EBLK, D_FEAT), jnp.float32),
          pltpu.VMEM((EBLK, D_FEAT), jnp.float32),
          pltpu.SemaphoreType.DMA,
          pltpu.SemaphoreType.DMA,
      ],
  )
  def gk(nf_hbm, snd_hbm, out_hbm, idx_v, buf_a, buf_b, sem_a, sem_b):
    wid = lax.axis_index("s") * NC + lax.axis_index("c")
    b0 = wid * GBLK_PER_TILE
    pltpu.sync_copy(snd_hbm.at[pl.ds(b0, GBLK_PER_TILE)], idx_v)

    def gstart(t, buf, sem):
      pltpu.async_copy(nf_hbm.at[idx_v.at[t]], buf, sem)

    def gwait(t, buf, sem):
      pltpu.make_async_copy(nf_hbm.at[idx_v.at[t]], buf, sem).wait()

    def wout(t, buf):
      pltpu.sync_copy(buf, out_hbm.at[pl.ds((b0 + t) * EBLK, EBLK)])

    gstart(0, buf_a, sem_a)

    def body(g, _):
      ta = 2 * g
      tb = 2 * g + 1
      gstart(tb, buf_b, sem_b)
      gwait(ta, buf_a, sem_a)
      wout(ta, buf_a)
      @pl.when(g < GBLK_PER_TILE // 2 - 1)
      def _():
        gstart(ta + 2, buf_a, sem_a)
      gwait(tb, buf_b, sem_b)
      wout(tb, buf_b)
      return ()

    lax.fori_loop(0, GBLK_PER_TILE // 2, body, ())

  return gk(node_feats, senders2d)


def _tc_radial(reT, W0, W1):
  isq8 = 1.0 / (8.0 ** 0.5)
  dn0 = (((0,), (0,)), ((), ()))

  def body(re_ref, w0_ref, w1_ref, out_ref):
    h = lax.dot_general(re_ref[...], w0_ref[...], dn0,
                        preferred_element_type=jnp.float32) * isq8
    h = h * jax.nn.sigmoid(h)
    h = jnp.dot(h, w1_ref[...], preferred_element_type=jnp.float32) * 0.125
    out_ref[...] = h * jax.nn.sigmoid(h)

  return pl.pallas_call(
      body,
      grid=(N_EDGES // EB_TC,),
      in_specs=[
          pl.BlockSpec((R_DIM, EB_TC), lambda i: (0, i)),
          pl.BlockSpec((R_DIM, HID), lambda i: (0, 0)),
          pl.BlockSpec((HID, HID), lambda i: (0, 0)),
      ],
      out_specs=pl.BlockSpec((EB_TC, HID), lambda i: (i, 0)),
      out_shape=jax.ShapeDtypeStruct((N_EDGES, HID), jnp.float32),
  )(reT, W0, W1)


def _tc_messages(msg, efT, hs, W2sub, R, Q, specs):
  dn0 = (((0,), (0,)), ((), ()))
  ncols = CCH * len(specs)

  def body(msg_ref, ef_ref, hs_ref, w2_ref, r_ref, q_ref, out_ref):
    w = jnp.dot(hs_ref[...], w2_ref[...],
                preferred_element_type=jnp.float32) * (0.125 * 0.25)
    m = msg_ref[...]
    if any(sp != "s" for sp in specs):
      erep = lax.dot_general(ef_ref[...], q_ref[...], dn0,
                             preferred_element_type=jnp.float32)
    for i, sp in enumerate(specs):
      lo = CCH * i
      if sp == "s":
        out_ref[:, lo:lo + CCH] = m * w[:, lo:lo + CCH]
      else:
        mrep = jnp.dot(m[:, 32 * sp:32 * sp + 32], r_ref[...],
                       preferred_element_type=jnp.float32)
        out_ref[:, lo:lo + CCH] = mrep * erep * w[:, lo:lo + CCH]

  return pl.pallas_call(
      body,
      grid=(N_EDGES // EB_TC,),
      in_specs=[
          pl.BlockSpec((EB_TC, D_FEAT), lambda i: (i, 0)),
          pl.BlockSpec((D_EDGE, EB_TC), lambda i: (0, i)),
          pl.BlockSpec((EB_TC, HID), lambda i: (i, 0)),
          pl.BlockSpec((HID, ncols), lambda i: (0, 0)),
          pl.BlockSpec((32, D_FEAT), lambda i: (0, 0)),
          pl.BlockSpec((D_EDGE, D_FEAT), lambda i: (0, 0)),
      ],
      out_specs=pl.BlockSpec((EB_TC, ncols), lambda i: (i, 0)),
      out_shape=jax.ShapeDtypeStruct((N_EDGES, ncols), jnp.float32),
  )(msg, efT, hs, W2sub, R, Q)


def _scatter_common(scratch_extra=()):
  return [
      pltpu.VMEM((88, EBLK), jnp.int32),
      pltpu.VMEM((8, EBLK), jnp.int32),
      pltpu.VMEM((EBLK, CCH), jnp.float32),
      pltpu.VMEM((EBLK, CCH), jnp.float32),
      pltpu.VMEM_SHARED((N_NODES, CCH), jnp.float32),
      pltpu.SemaphoreType.DMA,
      pltpu.SemaphoreType.DMA,
      pltpu.SemaphoreType.DMA,
      pltpu.SemaphoreType.DMA,
  ]


def _sc_scatter_pair(mp2, receivers2d, zeros_chunk):
  mesh = plsc.VectorSubcoreMesh(core_axis_name="c", subcore_axis_name="s")

  @functools.partial(
      pl.kernel,
      out_type=jax.ShapeDtypeStruct((N_NODES, 2 * CCH), jnp.float32),
      mesh=mesh,
      scratch_types=_scatter_common(),
  )
  def sk(mp_hbm, rcv_hbm, z_hbm, out_hbm, rcv_v, rcv_rem, u0, u1, acc,
         l0, l1, s0, s1):
    c = lax.axis_index("c")
    s = lax.axis_index("s")
    r0 = s * DRAIN_ROWS
    lrows = N_NODES - 15 * DRAIN_ROWS
    col = pl.multiple_of(c * CCH, CCH)

    base_b = s * SBLK_PER_TILE
    delta = lax.rem(base_b, 8)
    base_al = pl.multiple_of(base_b - delta, 8)
    pltpu.sync_copy(rcv_hbm.at[pl.ds(base_al, 88)], rcv_v)
    @pl.when(s == 0)
    def _():
      pltpu.sync_copy(rcv_hbm.at[pl.ds(16 * SBLK_PER_TILE, 8)], rcv_rem)

    ubufs = (u0, u1)
    lsems = (l0, l1)
    ssems = (s0, s1)

    def lstart(t, j):
      off = (base_b + t) * EBLK
      pltpu.async_copy(mp_hbm.at[pl.ds(off, EBLK), pl.ds(col, CCH)],
                       ubufs[j], lsems[j])

    def lwait(j):
      pltpu.make_async_copy(mp_hbm.at[pl.ds(0, EBLK), pl.ds(col, CCH)],
                            ubufs[j], lsems[j]).wait()

    def sstart(t, j):
      pltpu.async_copy(ubufs[j], acc.at[rcv_v.at[t + delta]], ssems[j],
                       add=True)

    def swait(t, j):
      pltpu.make_async_copy(ubufs[j], acc.at[rcv_v.at[t + delta]],
                            ssems[j]).wait()

    @pl.when(s < NS - 1)
    def _():
      pltpu.sync_copy(z_hbm.at[pl.ds(r0, DRAIN_ROWS)],
                      acc.at[pl.ds(r0, DRAIN_ROWS)])
    @pl.when(s == NS - 1)
    def _():
      pltpu.sync_copy(z_hbm.at[pl.ds(r0, lrows)], acc.at[pl.ds(r0, lrows)])
    plsc.subcore_barrier()

    @pl.when(s == 0)
    def _():
      for rr in range(2):
        off = (16 * SBLK_PER_TILE + rr) * EBLK
        pltpu.sync_copy(mp_hbm.at[pl.ds(off, EBLK), pl.ds(col, CCH)], u0)
        pltpu.sync_copy(u0, acc.at[rcv_rem.at[rr]], add=True)

    lstart(0, 0)

    def body(g, _):
      t = 2 * g
      lwait(0)
      sstart(t, 0)
      @pl.when(g > 0)
      def _():
        swait(t - 1, 1)
      lstart(t + 1, 1)
      lwait(1)
      sstart(t + 1, 1)
      swait(t, 0)
      @pl.when(g < SBLK_PER_TILE // 2 - 1)
      def _():
        lstart(t + 2, 0)
      return ()

    lax.fori_loop(0, SBLK_PER_TILE // 2, body, ())
    swait(SBLK_PER_TILE - 1, 1)

    plsc.subcore_barrier()
    @pl.when(s < NS - 1)
    def _():
      pltpu.sync_copy(acc.at[pl.ds(r0, DRAIN_ROWS)],
                      out_hbm.at[pl.ds(r0, DRAIN_ROWS), pl.ds(col, CCH)])
    @pl.when(s == NS - 1)
    def _():
      pltpu.sync_copy(acc.at[pl.ds(r0, lrows)],
                      out_hbm.at[pl.ds(r0, lrows), pl.ds(col, CCH)])
    plsc.subcore_barrier()

  return sk(mp2, receivers2d, zeros_chunk)


def _sc_scatter_half(mpc, receivers2d, zeros_chunk):
  mesh = plsc.VectorSubcoreMesh(core_axis_name="c", subcore_axis_name="s")

  @functools.partial(
      pl.kernel,
      out_type=(jax.ShapeDtypeStruct((N_NODES, CCH), jnp.float32),
                jax.ShapeDtypeStruct((N_NODES, CCH), jnp.float32)),
      mesh=mesh,
      scratch_types=_scatter_common(),
  )
  def sk(mp_hbm, rcv_hbm, z_hbm, outa_hbm, outb_hbm, rcv_v, rcv_rem, u0, u1,
         acc, l0, l1, s0, s1):
    c = lax.axis_index("c")
    s = lax.axis_index("s")
    r0 = s * DRAIN_ROWS
    lrows = N_NODES - 15 * DRAIN_ROWS

    npt = 39
    blk_lo = c * 625
    rem_lo = blk_lo + 624
    ring_n = 38

    base_b = blk_lo + s * npt
    delta = lax.rem(base_b, 8)
    base_al = pl.multiple_of(base_b - delta, 8)
    pltpu.sync_copy(rcv_hbm.at[pl.ds(base_al, 88)], rcv_v)
    rdelta = lax.rem(rem_lo, 8)
    rem_al = pl.multiple_of(rem_lo - rdelta, 8)
    @pl.when(s == 0)
    def _():
      pltpu.sync_copy(rcv_hbm.at[pl.ds(rem_al, 8)], rcv_rem)

    ubufs = (u0, u1)
    lsems = (l0, l1)
    ssems = (s0, s1)

    def lstart(t, j):
      off = (base_b + t) * EBLK
      pltpu.async_copy(mp_hbm.at[pl.ds(off, EBLK)], ubufs[j], lsems[j])

    def lwait(j):
      pltpu.make_async_copy(mp_hbm.at[pl.ds(0, EBLK)], ubufs[j],
                            lsems[j]).wait()

    def sstart(t, j):
      pltpu.async_copy(ubufs[j], acc.at[rcv_v.at[t + delta]], ssems[j],
                       add=True)

    def swait(t, j):
      pltpu.make_async_copy(ubufs[j], acc.at[rcv_v.at[t + delta]],
                            ssems[j]).wait()

    @pl.when(s < NS - 1)
    def _():
      pltpu.sync_copy(z_hbm.at[pl.ds(r0, DRAIN_ROWS)],
                      acc.at[pl.ds(r0, DRAIN_ROWS)])
    @pl.when(s == NS - 1)
    def _():
      pltpu.sync_copy(z_hbm.at[pl.ds(r0, lrows)], acc.at[pl.ds(r0, lrows)])
    plsc.subcore_barrier()

    @pl.when(s == 0)
    def _():
      off = rem_lo * EBLK
      pltpu.sync_copy(mp_hbm.at[pl.ds(off, EBLK)], u0)
      pltpu.sync_copy(u0, acc.at[rcv_rem.at[rdelta]], add=True)

    lstart(0, 0)

    def body(g, _):
      t = 2 * g
      lwait(0)
      sstart(t, 0)
      @pl.when(g > 0)
      def _():
        swait(t - 1, 1)
      lstart(t + 1, 1)
      lwait(1)
      sstart(t + 1, 1)
      swait(t, 0)
      @pl.when(g < ring_n // 2 - 1)
      def _():
        lstart(t + 2, 0)
      return ()

    lax.fori_loop(0, ring_n // 2, body, ())
    swait(ring_n - 1, 1)
    pltpu.sync_copy(mp_hbm.at[pl.ds((base_b + ring_n) * EBLK, EBLK)], u0)
    pltpu.sync_copy(u0, acc.at[rcv_v.at[ring_n + delta]], add=True)

    plsc.subcore_barrier()
    @pl.when(jnp.logical_and(c == 0, s < NS - 1))
    def _():
      pltpu.sync_copy(acc.at[pl.ds(r0, DRAIN_ROWS)],
                      outa_hbm.at[pl.ds(r0, DRAIN_ROWS)])
    @pl.when(jnp.logical_and(c == 0, s == NS - 1))
    def _():
      pltpu.sync_copy(acc.at[pl.ds(r0, lrows)], outa_hbm.at[pl.ds(r0, lrows)])
    @pl.when(jnp.logical_and(c == 1, s < NS - 1))
    def _():
      pltpu.sync_copy(acc.at[pl.ds(r0, DRAIN_ROWS)],
                      outb_hbm.at[pl.ds(r0, DRAIN_ROWS)])
    @pl.when(jnp.logical_and(c == 1, s == NS - 1))
    def _():
      pltpu.sync_copy(acc.at[pl.ds(r0, lrows)], outb_hbm.at[pl.ds(r0, lrows)])
    plsc.subcore_barrier()

  return sk(mpc, receivers2d, zeros_chunk)


def kernel(node_feats, edge_features, radial_embedding, senders, receivers,
           W0, W1, W2):
  senders = senders.astype(jnp.int32)
  receivers = receivers.astype(jnp.int32)
  R = jnp.repeat(jnp.eye(32, dtype=jnp.float32), D_EDGE, axis=1)
  Q = jnp.tile(jnp.eye(D_EDGE, dtype=jnp.float32), (1, 32))
  efT = edge_features.T
  reT = radial_embedding.T

  senders2d = jnp.pad(senders, (0, E_PAD - N_EDGES),
                      mode="wrap").reshape(-1, EBLK)
  receivers2d = jnp.pad(receivers.reshape(NBLK, EBLK),
                        ((0, RCV_PAD_BLKS - NBLK), (0, 0)))
  zeros_chunk = jnp.zeros((N_NODES, CCH), jnp.float32)

  w2A = jnp.concatenate([W2[:, 0:128], W2[:, 384:512]], axis=1)
  w2B = jnp.concatenate([W2[:, 128:256], W2[:, 512:640]], axis=1)
  w2C = W2[:, 256:384]

  hs = _tc_radial(reT, W0, W1)
  msg = _sc_gather(node_feats, senders2d)
  mpA = _tc_messages(msg, efT, hs, w2A, R, Q, ("s", 2))
  mpB = _tc_messages(msg, efT, hs, w2B, R, Q, (0, 3))
  mpC = _tc_messages(msg, efT, hs, w2C, R, Q, (1,))
  oA = _sc_scatter_pair(mpA, receivers2d, zeros_chunk)
  oB = _sc_scatter_pair(mpB, receivers2d, zeros_chunk)
  o2a, o2b = _sc_scatter_half(mpC, receivers2d, zeros_chunk)
  return jnp.concatenate(
      [oA[:, :CCH], oB[:, :CCH], o2a + o2b, oA[:, CCH:], oB[:, CCH:]],
      axis=1)


# speedup vs baseline: 1.0393x; 1.0393x over previous
import functools

import jax
import jax.numpy as jnp
from jax import lax
from jax.experimental import pallas as pl
from jax.experimental.pallas import tpu as pltpu
from jax.experimental.pallas import tpu_sc as plsc

N_NODES = 10000
N_EDGES = 160000
D_FEAT = 128
D_EDGE = 4
R_DIM = 8
HID = 64
N_IRR = D_FEAT * (1 + D_EDGE)

NC = 2
NS = 16
NW = NC * NS

EBLK = 128
NBLK = N_EDGES // EBLK
CCH = 128
NCH = N_IRR // CCH
DRAIN_ROWS = 632

GBLK_PER_TILE = 40
E_PAD = 32 * GBLK_PER_TILE * EBLK

SBLK_PER_TILE = 78
RCV_PAD_BLKS = 1312

EB_TC = 1280


def _sc_gather(node_feats, senders2d):
  mesh = plsc.VectorSubcoreMesh(core_axis_name="c", subcore_axis_name="s")

  @functools.partial(
      pl.kernel,
      out_type=jax.ShapeDtypeStruct((E_PAD, D_FEAT), jnp.float32),
      mesh=mesh,
      scratch_types=[
          pltpu.VMEM((GBLK_PER_TILE, EBLK), jnp.int32),
          pltpu.VMEM((EBLK, D_FEAT), jnp.float32),
          pltpu.VMEM((EBLK, D_FEAT), jnp.float32),
          pltpu.SemaphoreType.DMA,
          pltpu.SemaphoreType.DMA,
      ],
  )
  def gk(nf_hbm, snd_hbm, out_hbm, idx_v, buf_a, buf_b, sem_a, sem_b):
    wid = lax.axis_index("s") * NC + lax.axis_index("c")
    b0 = wid * GBLK_PER_TILE
    pltpu.sync_copy(snd_hbm.at[pl.ds(b0, GBLK_PER_TILE)], idx_v)

    def gstart(t, buf, sem):
      pltpu.async_copy(nf_hbm.at[idx_v.at[t]], buf, sem)

    def gwait(t, buf, sem):
      pltpu.make_async_copy(nf_hbm.at[idx_v.at[t]], buf, sem).wait()

    def wout(t, buf):
      pltpu.sync_copy(buf, out_hbm.at[pl.ds((b0 + t) * EBLK, EBLK)])

    gstart(0, buf_a, sem_a)

    def body(g, _):
      ta = 2 * g
      tb = 2 * g + 1
      gstart(tb, buf_b, sem_b)
      gwait(ta, buf_a, sem_a)
      wout(ta, buf_a)
      @pl.when(g < GBLK_PER_TILE // 2 - 1)
      def _():
        gstart(ta + 2, buf_a, sem_a)
      gwait(tb, buf_b, sem_b)
      wout(tb, buf_b)
      return ()

    lax.fori_loop(0, GBLK_PER_TILE // 2, body, ())

  return gk(node_feats, senders2d)


def _tc_radial(reT, W0, W1):
  isq8 = 1.0 / (8.0 ** 0.5)
  dn0 = (((0,), (0,)), ((), ()))

  def body(re_ref, w0_ref, w1_ref, out_ref):
    h = lax.dot_general(re_ref[...], w0_ref[...], dn0,
                        preferred_element_type=jnp.float32) * isq8
    h = h * jax.nn.sigmoid(h)
    h = jnp.dot(h, w1_ref[...], preferred_element_type=jnp.float32) * 0.125
    out_ref[...] = h * jax.nn.sigmoid(h)

  return pl.pallas_call(
      body,
      grid=(N_EDGES // EB_TC,),
      in_specs=[
          pl.BlockSpec((R_DIM, EB_TC), lambda i: (0, i)),
          pl.BlockSpec((R_DIM, HID), lambda i: (0, 0)),
          pl.BlockSpec((HID, HID), lambda i: (0, 0)),
      ],
      out_specs=pl.BlockSpec((EB_TC, HID), lambda i: (i, 0)),
      out_shape=jax.ShapeDtypeStruct((N_EDGES, HID), jnp.float32),
  )(reT, W0, W1)


def _tc_messages(msg, efT, hs, W2, R, Q):
  dn0 = (((0,), (0,)), ((), ()))

  def body(msg_ref, ef_ref, hs_ref, w2_ref, r_ref, q_ref, out_ref):
    w = jnp.dot(hs_ref[...], w2_ref[...],
                preferred_element_type=jnp.float32) * (0.125 * 0.25)
    m = msg_ref[...]
    erep = lax.dot_general(ef_ref[...], q_ref[...], dn0,
                           preferred_element_type=jnp.float32)
    out_ref[:, 0:D_FEAT] = m * w[:, 0:D_FEAT]
    for a in range(4):
      lo = D_FEAT * (1 + a)
      mrep = jnp.dot(m[:, 32 * a:32 * a + 32], r_ref[...],
                     preferred_element_type=jnp.float32)
      out_ref[:, lo:lo + D_FEAT] = mrep * erep * w[:, lo:lo + D_FEAT]

  grid = (N_EDGES // EB_TC,)
  return pl.pallas_call(
      body,
      grid=grid,
      in_specs=[
          pl.BlockSpec((EB_TC, D_FEAT), lambda i: (i, 0)),
          pl.BlockSpec((D_EDGE, EB_TC), lambda i: (0, i)),
          pl.BlockSpec((EB_TC, HID), lambda i: (i, 0)),
          pl.BlockSpec((HID, N_IRR), lambda i: (0, 0)),
          pl.BlockSpec((32, D_FEAT), lambda i: (0, 0)),
          pl.BlockSpec((D_EDGE, D_FEAT), lambda i: (0, 0)),
      ],
      out_specs=pl.BlockSpec((EB_TC, N_IRR), lambda i: (i, 0)),
      out_shape=jax.ShapeDtypeStruct((N_EDGES, N_IRR), jnp.float32),
  )(msg, efT, hs, W2, R, Q)


def _sc_scatter(mp, receivers2d, zeros_chunk):
  mesh = plsc.VectorSubcoreMesh(core_axis_name="c", subcore_axis_name="s")

  @functools.partial(
      pl.kernel,
      out_type=(jax.ShapeDtypeStruct((N_NODES, N_IRR), jnp.float32),
                jax.ShapeDtypeStruct((N_NODES, CCH), jnp.float32)),
      mesh=mesh,
      scratch_types=[
          pltpu.VMEM((88, EBLK), jnp.int32),
          pltpu.VMEM((8, EBLK), jnp.int32),
          pltpu.VMEM((EBLK, CCH), jnp.float32),
          pltpu.VMEM((EBLK, CCH), jnp.float32),
          pltpu.VMEM_SHARED((N_NODES, CCH), jnp.float32),
          pltpu.SemaphoreType.DMA,
          pltpu.SemaphoreType.DMA,
          pltpu.SemaphoreType.DMA,
          pltpu.SemaphoreType.DMA,
      ],
  )
  def sk(mp_hbm, rcv_hbm, z_hbm, out_hbm, part_hbm, rcv_v, rcv_rem, u0, u1,
         acc, l0, l1, s0, s1):
    c = lax.axis_index("c")
    s = lax.axis_index("s")

    r0 = s * DRAIN_ROWS
    lrows = N_NODES - 15 * DRAIN_ROWS

    for k in range(3):
      if k < 2:
        q = c * 3 + k
        npt = SBLK_PER_TILE
        blk_lo = 0
        rem_lo = 16 * SBLK_PER_TILE
        nrem = 2
        ring_n = SBLK_PER_TILE
      else:
        q = c * 0 + 2
        npt = 39
        blk_lo = c * 625
        rem_lo = blk_lo + 624
        nrem = 1
        ring_n = 38
      col = pl.multiple_of(q * CCH, CCH)

      base_b = blk_lo + s * npt
      delta = lax.rem(base_b, 8)
      base_al = pl.multiple_of(base_b - delta, 8)
      pltpu.sync_copy(rcv_hbm.at[pl.ds(base_al, 88)], rcv_v)
      rdelta = lax.rem(rem_lo, 8)
      rem_al = pl.multiple_of(rem_lo - rdelta, 8)
      @pl.when(s == 0)
      def _():
        pltpu.sync_copy(rcv_hbm.at[pl.ds(rem_al, 8)], rcv_rem)

      ubufs = (u0, u1)
      lsems = (l0, l1)
      ssems = (s0, s1)

      def lstart(t, j):
        off = (base_b + t) * EBLK
        pltpu.async_copy(mp_hbm.at[pl.ds(off, EBLK), pl.ds(col, CCH)],
                         ubufs[j], lsems[j])

      def lwait(j):
        pltpu.make_async_copy(mp_hbm.at[pl.ds(0, EBLK), pl.ds(col, CCH)],
                              ubufs[j], lsems[j]).wait()

      def sstart(t, j):
        pltpu.async_copy(ubufs[j], acc.at[rcv_v.at[t + delta]], ssems[j],
                         add=True)

      def swait(t, j):
        pltpu.make_async_copy(ubufs[j], acc.at[rcv_v.at[t + delta]],
                              ssems[j]).wait()

      @pl.when(s < NS - 1)
      def _():
        pltpu.sync_copy(z_hbm.at[pl.ds(r0, DRAIN_ROWS)],
                        acc.at[pl.ds(r0, DRAIN_ROWS)])
      @pl.when(s == NS - 1)
      def _():
        pltpu.sync_copy(z_hbm.at[pl.ds(r0, lrows)], acc.at[pl.ds(r0, lrows)])
      plsc.subcore_barrier()

      @pl.when(s == 0)
      def _():
        for rr in range(nrem):
          off = (rem_lo + rr) * EBLK
          pltpu.sync_copy(mp_hbm.at[pl.ds(off, EBLK), pl.ds(col, CCH)], u0)
          pltpu.sync_copy(u0, acc.at[rcv_rem.at[rdelta + rr]], add=True)

      lstart(0, 0)

      def body(g, _):
        t = 2 * g
        lwait(0)
        sstart(t, 0)
        @pl.when(g > 0)
        def _():
          swait(t - 1, 1)
        lstart(t + 1, 1)
        lwait(1)
        sstart(t + 1, 1)
        swait(t, 0)
        @pl.when(g < ring_n // 2 - 1)
        def _():
          lstart(t + 2, 0)
        return ()

      lax.fori_loop(0, ring_n // 2, body, ())
      swait(ring_n - 1, 1)
      if ring_n < npt:
        pltpu.sync_copy(mp_hbm.at[pl.ds((base_b + ring_n) * EBLK, EBLK),
                                  pl.ds(col, CCH)], u0)
        pltpu.sync_copy(u0, acc.at[rcv_v.at[ring_n + delta]], add=True)

      plsc.subcore_barrier()
      if k < 2:
        @pl.when(s < NS - 1)
        def _():
          pltpu.sync_copy(acc.at[pl.ds(r0, DRAIN_ROWS)],
                          out_hbm.at[pl.ds(r0, DRAIN_ROWS), pl.ds(col, CCH)])
        @pl.when(s == NS - 1)
        def _():
          pltpu.sync_copy(acc.at[pl.ds(r0, lrows)],
                          out_hbm.at[pl.ds(r0, lrows), pl.ds(col, CCH)])
      else:
        @pl.when(jnp.logical_and(c == 0, s < NS - 1))
        def _():
          pltpu.sync_copy(acc.at[pl.ds(r0, DRAIN_ROWS)],
                          out_hbm.at[pl.ds(r0, DRAIN_ROWS), pl.ds(col, CCH)])
        @pl.when(jnp.logical_and(c == 0, s == NS - 1))
        def _():
          pltpu.sync_copy(acc.at[pl.ds(r0, lrows)],
                          out_hbm.at[pl.ds(r0, lrows), pl.ds(col, CCH)])
        @pl.when(jnp.logical_and(c == 1, s < NS - 1))
        def _():
          pltpu.sync_copy(acc.at[pl.ds(r0, DRAIN_ROWS)],
                          part_hbm.at[pl.ds(r0, DRAIN_ROWS)])
        @pl.when(jnp.logical_and(c == 1, s == NS - 1))
        def _():
          pltpu.sync_copy(acc.at[pl.ds(r0, lrows)],
                          part_hbm.at[pl.ds(r0, lrows)])
      plsc.subcore_barrier()

  return sk(mp, receivers2d, zeros_chunk)


def kernel(node_feats, edge_features, radial_embedding, senders, receivers,
           W0, W1, W2):
  senders = senders.astype(jnp.int32)
  receivers = receivers.astype(jnp.int32)
  R = jnp.repeat(jnp.eye(32, dtype=jnp.float32), D_EDGE, axis=1)
  Q = jnp.tile(jnp.eye(D_EDGE, dtype=jnp.float32), (1, 32))

  senders2d = jnp.pad(senders, (0, E_PAD - N_EDGES),
                      mode="wrap").reshape(-1, EBLK)
  receivers2d = jnp.pad(receivers.reshape(NBLK, EBLK),
                        ((0, RCV_PAD_BLKS - NBLK), (0, 0)))

  hs = _tc_radial(radial_embedding.T, W0, W1)
  msg = _sc_gather(node_feats, senders2d)
  mp = _tc_messages(msg, edge_features.T, hs, W2, R, Q)
  zeros_chunk = jnp.zeros((N_NODES, CCH), jnp.float32)
  main, part = _sc_scatter(mp, receivers2d, zeros_chunk)
  return main.at[:, 2 * CCH:3 * CCH].add(part)


# speedup vs baseline: 1.1455x; 1.1022x over previous
import functools

import jax
import jax.numpy as jnp
from jax import lax
from jax.experimental import pallas as pl
from jax.experimental.pallas import tpu as pltpu
from jax.experimental.pallas import tpu_sc as plsc

N_NODES = 10000
N_EDGES = 160000
D_FEAT = 128
D_EDGE = 4
R_DIM = 8
HID = 64
N_IRR = D_FEAT * (1 + D_EDGE)

NC = 2
NS = 16
NW = NC * NS

EBLK = 128
NBLK = N_EDGES // EBLK
CCH = 128
NCH = N_IRR // CCH
DRAIN_ROWS = 632

GBLK_PER_TILE = 40
E_PAD = 32 * GBLK_PER_TILE * EBLK

SBLK_PER_TILE = 78
RCV_PAD_BLKS = 1312

EB_TC = 1280


def _sc_gather(node_feats, senders2d):
  mesh = plsc.VectorSubcoreMesh(core_axis_name="c", subcore_axis_name="s")

  @functools.partial(
      pl.kernel,
      out_type=jax.ShapeDtypeStruct((E_PAD, D_FEAT), jnp.float32),
      mesh=mesh,
      scratch_types=[
          pltpu.VMEM((GBLK_PER_TILE, EBLK), jnp.int32),
          pltpu.VMEM((EBLK, D_FEAT), jnp.float32),
          pltpu.VMEM((EBLK, D_FEAT), jnp.float32),
          pltpu.SemaphoreType.DMA,
          pltpu.SemaphoreType.DMA,
      ],
  )
  def gk(nf_hbm, snd_hbm, out_hbm, idx_v, buf_a, buf_b, sem_a, sem_b):
    wid = lax.axis_index("s") * NC + lax.axis_index("c")
    b0 = wid * GBLK_PER_TILE
    pltpu.sync_copy(snd_hbm.at[pl.ds(b0, GBLK_PER_TILE)], idx_v)

    def gstart(t, buf, sem):
      pltpu.async_copy(nf_hbm.at[idx_v.at[t]], buf, sem)

    def gwait(t, buf, sem):
      pltpu.make_async_copy(nf_hbm.at[idx_v.at[t]], buf, sem).wait()

    def wout(t, buf):
      pltpu.sync_copy(buf, out_hbm.at[pl.ds((b0 + t) * EBLK, EBLK)])

    gstart(0, buf_a, sem_a)

    def body(g, _):
      ta = 2 * g
      tb = 2 * g + 1
      gstart(tb, buf_b, sem_b)
      gwait(ta, buf_a, sem_a)
      wout(ta, buf_a)
      @pl.when(g < GBLK_PER_TILE // 2 - 1)
      def _():
        gstart(ta + 2, buf_a, sem_a)
      gwait(tb, buf_b, sem_b)
      wout(tb, buf_b)
      return ()

    lax.fori_loop(0, GBLK_PER_TILE // 2, body, ())

  return gk(node_feats, senders2d)


def _tc_messages(msg, edge_features, radial_embedding, W0, W1, W2, R, Q):
  isq8 = 1.0 / (8.0 ** 0.5)

  dn0 = (((0,), (0,)), ((), ()))

  def body(msg_ref, ef_ref, re_ref, w0_ref, w1_ref, w2_ref, r_ref, q_ref,
           out_ref):
    h = lax.dot_general(re_ref[...], w0_ref[...], dn0,
                        preferred_element_type=jnp.float32) * isq8
    h = h * jax.nn.sigmoid(h)
    h = jnp.dot(h, w1_ref[...], preferred_element_type=jnp.float32) * 0.125
    h = h * jax.nn.sigmoid(h)
    w = jnp.dot(h, w2_ref[...], preferred_element_type=jnp.float32) * (0.125 * 0.25)
    m = msg_ref[...]
    erep = lax.dot_general(ef_ref[...], q_ref[...], dn0,
                           preferred_element_type=jnp.float32)
    out_ref[:, 0:D_FEAT] = m * w[:, 0:D_FEAT]
    for a in range(4):
      lo = D_FEAT * (1 + a)
      mrep = jnp.dot(m[:, 32 * a:32 * a + 32], r_ref[...],
                     preferred_element_type=jnp.float32)
      out_ref[:, lo:lo + D_FEAT] = mrep * erep * w[:, lo:lo + D_FEAT]

  grid = (N_EDGES // EB_TC,)
  return pl.pallas_call(
      body,
      grid=grid,
      in_specs=[
          pl.BlockSpec((EB_TC, D_FEAT), lambda i: (i, 0)),
          pl.BlockSpec((D_EDGE, EB_TC), lambda i: (0, i)),
          pl.BlockSpec((R_DIM, EB_TC), lambda i: (0, i)),
          pl.BlockSpec((R_DIM, HID), lambda i: (0, 0)),
          pl.BlockSpec((HID, HID), lambda i: (0, 0)),
          pl.BlockSpec((HID, N_IRR), lambda i: (0, 0)),
          pl.BlockSpec((32, D_FEAT), lambda i: (0, 0)),
          pl.BlockSpec((D_EDGE, D_FEAT), lambda i: (0, 0)),
      ],
      out_specs=pl.BlockSpec((EB_TC, N_IRR), lambda i: (i, 0)),
      out_shape=jax.ShapeDtypeStruct((N_EDGES, N_IRR), jnp.float32),
  )(msg, edge_features.T, radial_embedding.T, W0, W1, W2, R, Q)


def _sc_scatter(mp, receivers2d, zeros_chunk):
  mesh = plsc.VectorSubcoreMesh(core_axis_name="c", subcore_axis_name="s")

  @functools.partial(
      pl.kernel,
      out_type=(jax.ShapeDtypeStruct((N_NODES, N_IRR), jnp.float32),
                jax.ShapeDtypeStruct((N_NODES, CCH), jnp.float32)),
      mesh=mesh,
      scratch_types=[
          pltpu.VMEM((88, EBLK), jnp.int32),
          pltpu.VMEM((8, EBLK), jnp.int32),
          pltpu.VMEM((EBLK, CCH), jnp.float32),
          pltpu.VMEM((EBLK, CCH), jnp.float32),
          pltpu.VMEM_SHARED((N_NODES, CCH), jnp.float32),
          pltpu.SemaphoreType.DMA,
          pltpu.SemaphoreType.DMA,
          pltpu.SemaphoreType.DMA,
          pltpu.SemaphoreType.DMA,
      ],
  )
  def sk(mp_hbm, rcv_hbm, z_hbm, out_hbm, part_hbm, rcv_v, rcv_rem, u0, u1,
         acc, l0, l1, s0, s1):
    c = lax.axis_index("c")
    s = lax.axis_index("s")

    r0 = s * DRAIN_ROWS
    lrows = N_NODES - 15 * DRAIN_ROWS

    for k in range(3):
      if k < 2:
        q = c * 3 + k
        npt = SBLK_PER_TILE
        blk_lo = 0
        rem_lo = 16 * SBLK_PER_TILE
        nrem = 2
        ring_n = SBLK_PER_TILE
      else:
        q = c * 0 + 2
        npt = 39
        blk_lo = c * 625
        rem_lo = blk_lo + 624
        nrem = 1
        ring_n = 38
      col = pl.multiple_of(q * CCH, CCH)

      base_b = blk_lo + s * npt
      delta = lax.rem(base_b, 8)
      base_al = pl.multiple_of(base_b - delta, 8)
      pltpu.sync_copy(rcv_hbm.at[pl.ds(base_al, 88)], rcv_v)
      rdelta = lax.rem(rem_lo, 8)
      rem_al = pl.multiple_of(rem_lo - rdelta, 8)
      @pl.when(s == 0)
      def _():
        pltpu.sync_copy(rcv_hbm.at[pl.ds(rem_al, 8)], rcv_rem)

      ubufs = (u0, u1)
      lsems = (l0, l1)
      ssems = (s0, s1)

      def lstart(t, j):
        off = (base_b + t) * EBLK
        pltpu.async_copy(mp_hbm.at[pl.ds(off, EBLK), pl.ds(col, CCH)],
                         ubufs[j], lsems[j])

      def lwait(j):
        pltpu.make_async_copy(mp_hbm.at[pl.ds(0, EBLK), pl.ds(col, CCH)],
                              ubufs[j], lsems[j]).wait()

      def sstart(t, j):
        pltpu.async_copy(ubufs[j], acc.at[rcv_v.at[t + delta]], ssems[j],
                         add=True)

      def swait(t, j):
        pltpu.make_async_copy(ubufs[j], acc.at[rcv_v.at[t + delta]],
                              ssems[j]).wait()

      @pl.when(s < NS - 1)
      def _():
        pltpu.sync_copy(z_hbm.at[pl.ds(r0, DRAIN_ROWS)],
                        acc.at[pl.ds(r0, DRAIN_ROWS)])
      @pl.when(s == NS - 1)
      def _():
        pltpu.sync_copy(z_hbm.at[pl.ds(r0, lrows)], acc.at[pl.ds(r0, lrows)])
      plsc.subcore_barrier()

      @pl.when(s == 0)
      def _():
        for rr in range(nrem):
          off = (rem_lo + rr) * EBLK
          pltpu.sync_copy(mp_hbm.at[pl.ds(off, EBLK), pl.ds(col, CCH)], u0)
          pltpu.sync_copy(u0, acc.at[rcv_rem.at[rdelta + rr]], add=True)

      lstart(0, 0)

      def body(g, _):
        t = 2 * g
        lwait(0)
        sstart(t, 0)
        @pl.when(g > 0)
        def _():
          swait(t - 1, 1)
        lstart(t + 1, 1)
        lwait(1)
        sstart(t + 1, 1)
        swait(t, 0)
        @pl.when(g < ring_n // 2 - 1)
        def _():
          lstart(t + 2, 0)
        return ()

      lax.fori_loop(0, ring_n // 2, body, ())
      swait(ring_n - 1, 1)
      if ring_n < npt:
        pltpu.sync_copy(mp_hbm.at[pl.ds((base_b + ring_n) * EBLK, EBLK),
                                  pl.ds(col, CCH)], u0)
        pltpu.sync_copy(u0, acc.at[rcv_v.at[ring_n + delta]], add=True)

      plsc.subcore_barrier()
      if k < 2:
        @pl.when(s < NS - 1)
        def _():
          pltpu.sync_copy(acc.at[pl.ds(r0, DRAIN_ROWS)],
                          out_hbm.at[pl.ds(r0, DRAIN_ROWS), pl.ds(col, CCH)])
        @pl.when(s == NS - 1)
        def _():
          pltpu.sync_copy(acc.at[pl.ds(r0, lrows)],
                          out_hbm.at[pl.ds(r0, lrows), pl.ds(col, CCH)])
      else:
        @pl.when(jnp.logical_and(c == 0, s < NS - 1))
        def _():
          pltpu.sync_copy(acc.at[pl.ds(r0, DRAIN_ROWS)],
                          out_hbm.at[pl.ds(r0, DRAIN_ROWS), pl.ds(col, CCH)])
        @pl.when(jnp.logical_and(c == 0, s == NS - 1))
        def _():
          pltpu.sync_copy(acc.at[pl.ds(r0, lrows)],
                          out_hbm.at[pl.ds(r0, lrows), pl.ds(col, CCH)])
        @pl.when(jnp.logical_and(c == 1, s < NS - 1))
        def _():
          pltpu.sync_copy(acc.at[pl.ds(r0, DRAIN_ROWS)],
                          part_hbm.at[pl.ds(r0, DRAIN_ROWS)])
        @pl.when(jnp.logical_and(c == 1, s == NS - 1))
        def _():
          pltpu.sync_copy(acc.at[pl.ds(r0, lrows)],
                          part_hbm.at[pl.ds(r0, lrows)])
      plsc.subcore_barrier()

  return sk(mp, receivers2d, zeros_chunk)


def kernel(node_feats, edge_features, radial_embedding, senders, receivers,
           W0, W1, W2):
  senders = senders.astype(jnp.int32)
  receivers = receivers.astype(jnp.int32)
  R = jnp.repeat(jnp.eye(32, dtype=jnp.float32), D_EDGE, axis=1)
  Q = jnp.tile(jnp.eye(D_EDGE, dtype=jnp.float32), (1, 32))

  senders2d = jnp.pad(senders, (0, E_PAD - N_EDGES),
                      mode="wrap").reshape(-1, EBLK)
  receivers2d = jnp.pad(receivers.reshape(NBLK, EBLK),
                        ((0, RCV_PAD_BLKS - NBLK), (0, 0)))

  msg = _sc_gather(node_feats, senders2d)
  mp = _tc_messages(msg, edge_features, radial_embedding, W0, W1, W2, R, Q)
  zeros_chunk = jnp.zeros((N_NODES, CCH), jnp.float32)
  main, part = _sc_scatter(mp, receivers2d, zeros_chunk)
  return main.at[:, 2 * CCH:3 * CCH].add(part)
